# baseline (speedup 1.0000x reference)
import functools

import jax
import jax.numpy as jnp
from jax import lax
from jax.experimental import pallas as pl
from jax.experimental.pallas import tpu as pltpu
from jax.experimental.pallas import tpu_sc as plsc

B, V, D, H = 16384, 100000, 64, 512


TBV = 6400
NTB = 8
K = NTB * TBV


def _tpose_body(lo_ref, hi_ref, out_ref):
    out_ref[...] = jnp.concatenate([lo_ref[...].T, hi_ref[...].T], axis=-1)


def _tpose(tt):
    return pl.pallas_call(
        _tpose_body,
        grid=(NTB,),
        in_specs=[
            pl.BlockSpec((D, TBV), lambda i: (0, i)),
            pl.BlockSpec((D, TBV), lambda i: (0, i + NTB)),
        ],
        out_specs=pl.BlockSpec((TBV, 128), lambda i: (i, 0)),
        out_shape=jax.ShapeDtypeStruct((K, 128), jnp.float32),
        compiler_params=pltpu.CompilerParams(
            dimension_semantics=("parallel",)),
    )(tt, tt)


def _make_sc_gather():
    info = plsc.get_sparse_core_info()
    NC, NS = info.num_cores, info.num_subcores
    NW = NC * NS
    b_per_w = B // NW
    mesh = plsc.VectorSubcoreMesh(core_axis_name="c", subcore_axis_name="s")

    @functools.partial(
        pl.kernel,
        mesh=mesh,
        out_type=jax.ShapeDtypeStruct((B, 128), jnp.float32),
        scratch_types=[
            pltpu.VMEM((b_per_w,), jnp.int32),
            pltpu.VMEM((b_per_w, 128), jnp.float32),
            pltpu.SemaphoreType.DMA,
        ],
        compiler_params=pltpu.CompilerParams(use_tc_tiling_on_sc=False),
    )
    def gather_kernel(idx_hbm, tp_hbm, out_hbm, idx_v, rows_v, sem):
        wid = lax.axis_index("s") * NC + lax.axis_index("c")
        base = wid * b_per_w
        pltpu.sync_copy(idx_hbm.at[pl.ds(base, b_per_w)], idx_v)
        pltpu.async_copy(tp_hbm.at[idx_v], rows_v, sem).wait()
        pltpu.sync_copy(rows_v, out_hbm.at[pl.ds(base, b_per_w)])

    return gather_kernel


_sc_gather = _make_sc_gather()


BK = 1024


def _mlp_body(e_ref, par_ref, W1_ref, b1_ref, W2_ref, b2_ref, w3_ref, b3_ref,
              out_ref):
    lane_hi = lax.broadcasted_iota(jnp.int32, (BK, 128), 1) >= D
    want_hi = par_ref[...] > 0.5
    e = jnp.where(lane_hi == want_hi, e_ref[...], 0.0).astype(jnp.bfloat16)
    h = jnp.tanh(
        jax.lax.dot_general(e, W1_ref[...], (((1,), (0,)), ((), ())),
                            preferred_element_type=jnp.float32)
        + b1_ref[...]).astype(jnp.bfloat16)
    h = jnp.tanh(
        jax.lax.dot_general(h, W2_ref[...], (((1,), (0,)), ((), ())),
                            preferred_element_type=jnp.float32)
        + b2_ref[...])
    res = jnp.sum(h * w3_ref[...], axis=1, keepdims=True) + b3_ref[...]
    out_ref[...] = res.reshape(BK // 128, 128)


def _mlp(e, par, W1s, b1, W2, b2, w3, b3):
    grid = (B // BK,)
    return pl.pallas_call(
        _mlp_body,
        grid=grid,
        in_specs=[
            pl.BlockSpec((BK, 128), lambda i: (i, 0)),
            pl.BlockSpec((BK, 1), lambda i: (i, 0)),
            pl.BlockSpec((128, H), lambda i: (0, 0)),
            pl.BlockSpec((1, H), lambda i: (0, 0)),
            pl.BlockSpec((H, H), lambda i: (0, 0)),
            pl.BlockSpec((1, H), lambda i: (0, 0)),
            pl.BlockSpec((1, H), lambda i: (0, 0)),
            pl.BlockSpec((1, 1), lambda i: (0, 0)),
        ],
        out_specs=pl.BlockSpec((BK // 128, 128), lambda i: (i, 0)),
        out_shape=jax.ShapeDtypeStruct((B // 128, 128), jnp.float32),
        compiler_params=pltpu.CompilerParams(
            dimension_semantics=("parallel",)),
    )(e, par, W1s, b1, W2, b2, w3, b3)


def kernel(x, table, W1, b1, W2, b2, W3, b3):
    idx = jnp.reshape(x, (B,)).astype(jnp.int32)
    hi = (idx >= K).astype(jnp.int32)
    par = hi.astype(jnp.float32).reshape(B, 1)
    idx2 = idx - hi * K
    tp = _tpose(table.T)
    e = _sc_gather(idx2, tp)
    W1s = jnp.concatenate([W1, W1], axis=0).astype(jnp.bfloat16)
    out = _mlp(e, par, W1s, b1.reshape(1, H), W2.astype(jnp.bfloat16),
               b2.reshape(1, H), W3.reshape(1, H), b3.reshape(1, 1))
    return out.reshape(B, 1)


# speedup vs baseline: 1.0830x; 1.0830x over previous
import functools

import jax
import jax.numpy as jnp
from jax import lax
from jax.experimental import pallas as pl
from jax.experimental.pallas import tpu as pltpu
from jax.experimental.pallas import tpu_sc as plsc

B, V, D, H = 16384, 100000, 64, 512


TBV = 6400
NTB = (V + TBV - 1) // TBV


def _tpose_body(tt_ref, out_ref):
    xt = tt_ref[...].T
    out_ref[...] = jnp.concatenate([xt, jnp.zeros_like(xt)], axis=-1)


def _tpose(tt):
    return pl.pallas_call(
        _tpose_body,
        grid=(NTB,),
        in_specs=[pl.BlockSpec((D, TBV), lambda i: (0, i))],
        out_specs=pl.BlockSpec((TBV, 128), lambda i: (i, 0)),
        out_shape=jax.ShapeDtypeStruct((V, 128), jnp.float32),
    )(tt)


def _make_sc_gather():
    info = plsc.get_sparse_core_info()
    NC, NS = info.num_cores, info.num_subcores
    NW = NC * NS
    b_per_w = B // NW
    mesh = plsc.VectorSubcoreMesh(core_axis_name="c", subcore_axis_name="s")

    @functools.partial(
        pl.kernel,
        mesh=mesh,
        out_type=jax.ShapeDtypeStruct((B, 128), jnp.float32),
        scratch_types=[
            pltpu.VMEM((b_per_w,), jnp.int32),
            pltpu.VMEM((b_per_w, 128), jnp.float32),
            pltpu.SemaphoreType.DMA,
        ],
        compiler_params=pltpu.CompilerParams(use_tc_tiling_on_sc=False),
    )
    def gather_kernel(idx_hbm, tp_hbm, out_hbm, idx_v, rows_v, sem):
        wid = lax.axis_index("s") * NC + lax.axis_index("c")
        base = wid * b_per_w
        pltpu.sync_copy(idx_hbm.at[pl.ds(base, b_per_w)], idx_v)
        pltpu.async_copy(tp_hbm.at[idx_v], rows_v, sem).wait()
        pltpu.sync_copy(rows_v, out_hbm.at[pl.ds(base, b_per_w)])

    return gather_kernel


_sc_gather = _make_sc_gather()


BK = 1024


def _mlp_body(e_ref, W1_ref, b1_ref, W2_ref, b2_ref, W3_ref, b3_ref, out_ref):
    h = jnp.tanh(
        jax.lax.dot_general(e_ref[...], W1_ref[...], (((1,), (0,)), ((), ())),
                            preferred_element_type=jnp.float32)
        + b1_ref[...])
    h = jnp.tanh(
        jax.lax.dot_general(h, W2_ref[...], (((1,), (0,)), ((), ())),
                            preferred_element_type=jnp.float32)
        + b2_ref[...])
    out_ref[...] = (
        jax.lax.dot_general(h, W3_ref[...], (((1,), (0,)), ((), ())),
                            preferred_element_type=jnp.float32)
        + b3_ref[...])


def _mlp(e, W1p, b1, W2, b2, W3, b3):
    grid = (B // BK,)
    return pl.pallas_call(
        _mlp_body,
        grid=grid,
        in_specs=[
            pl.BlockSpec((BK, 128), lambda i: (i, 0)),
            pl.BlockSpec((128, H), lambda i: (0, 0)),
            pl.BlockSpec((1, H), lambda i: (0, 0)),
            pl.BlockSpec((H, H), lambda i: (0, 0)),
            pl.BlockSpec((1, H), lambda i: (0, 0)),
            pl.BlockSpec((H, 1), lambda i: (0, 0)),
            pl.BlockSpec((1, 1), lambda i: (0, 0)),
        ],
        out_specs=pl.BlockSpec((BK, 1), lambda i: (i, 0)),
        out_shape=jax.ShapeDtypeStruct((B, 1), jnp.float32),
    )(e, W1p, b1, W2, b2, W3, b3)


def kernel(x, table, W1, b1, W2, b2, W3, b3):
    idx = jnp.reshape(x, (B,)).astype(jnp.int32)
    tp = _tpose(table.T)
    e = _sc_gather(idx, tp)
    W1p = jnp.pad(W1, ((0, 128 - D), (0, 0)))
    return _mlp(e, W1p, b1.reshape(1, H), W2, b2.reshape(1, H),
                W3, b3.reshape(1, 1))
